# baseline (speedup 1.0000x reference)
import functools

import jax
import jax.numpy as jnp
from jax import lax
from jax.experimental import pallas as pl
from jax.experimental.pallas import tpu as pltpu
from jax.experimental.pallas import tpu_sc as plsc

BT = 256
E = 8
D = 2048
I = 2048
TOPK = 2
NP = BT * TOPK
TN = 1024
NT = I // TN
TND = 512
ND = D // TND
S = NT + ND

NTILES = 32
TPT = BT // NTILES

_MESH = plsc.VectorSubcoreMesh(core_axis_name="c", subcore_axis_name="s")


def _wid():
    return lax.axis_index("s") * 2 + lax.axis_index("c")


def _moe_body(x_ref, ert_ref, rw_ref, wu_ref, wg_ref, w2_ref, y_ref, h_ref):
    e = pl.program_id(0)
    s = pl.program_id(1)

    @pl.when(s < NT)
    def _():
        x = x_ref[...]
        u = jax.lax.dot_general(x, wu_ref[0], (((1,), (1,)), ((), ())),
                                preferred_element_type=jnp.float32)
        g = jax.lax.dot_general(x, wg_ref[0], (((1,), (1,)), ((), ())),
                                preferred_element_type=jnp.float32)
        h_ref[:, pl.ds(s * TN, TN)] = (u * jax.nn.sigmoid(u)) * g

    @pl.when(s >= NT)
    def _():
        y = jax.lax.dot_general(h_ref[...], w2_ref[0], (((1,), (1,)), ((), ())),
                                preferred_element_type=jnp.float32)
        sel = (ert_ref[...] == e).astype(jnp.float32) * rw_ref[...]
        we = jnp.sum(sel, axis=1, keepdims=True)
        y_ref[0] = y * we


@functools.partial(
    pl.kernel,
    out_type=jax.ShapeDtypeStruct((BT, D), jnp.float32),
    mesh=_MESH,
    compiler_params=pltpu.CompilerParams(needs_layout_passes=False),
    scratch_types=[
        pltpu.VMEM((16,), jnp.int32),
        pltpu.VMEM((TPT,), jnp.int32),
        pltpu.VMEM((TPT,), jnp.int32),
        pltpu.VMEM((TPT, D), jnp.float32),
        pltpu.VMEM((TPT, D), jnp.float32),
        pltpu.SemaphoreType.DMA,
        pltpu.SemaphoreType.DMA,
    ],
)
def _combine(ysc_hbm, keys_hbm, out_hbm, k_v, ia_v, ib_v, ra_v, rb_v, sa, sb):
    base = _wid() * TPT
    pltpu.sync_copy(keys_hbm.at[pl.ds(base * TOPK, 16)], k_v)
    iota = lax.iota(jnp.int32, 16)
    ones = jnp.ones((16,), jnp.int32)
    half = lax.shift_right_logical(iota, ones)
    tok = jnp.full((16,), base, jnp.int32) + half
    pc = k_v[...] * jnp.full((16,), BT, jnp.int32) + tok
    even = jnp.bitwise_and(iota, ones) == 0
    plsc.store_scatter(ia_v, [half], pc, mask=even)
    plsc.store_scatter(ib_v, [half], pc, mask=jnp.logical_not(even))
    ca = pltpu.async_copy(ysc_hbm.at[ia_v], ra_v, sa)
    cb = pltpu.async_copy(ysc_hbm.at[ib_v], rb_v, sb)
    ca.wait()
    cb.wait()

    def add_body(j, _):
        sl = pl.ds(j * 16, 16)
        for r in range(TPT):
            ra_v[r, sl] = ra_v[r, sl] + rb_v[r, sl]
        return 0

    lax.fori_loop(0, D // 16, add_body, 0)
    pltpu.sync_copy(ra_v, out_hbm.at[pl.ds(base, TPT)])


def kernel(hidden_states, expert_routing_table, router_weights, w13_weight, w2_weight):
    x = hidden_states.astype(jnp.float32)
    ert = expert_routing_table.astype(jnp.int32)
    rw = router_weights.astype(jnp.float32)

    ysc = pl.pallas_call(
        _moe_body,
        grid=(E, S),
        in_specs=[
            pl.BlockSpec((BT, D), lambda e, s: (0, 0)),
            pl.BlockSpec((BT, 2), lambda e, s: (0, 0)),
            pl.BlockSpec((BT, 2), lambda e, s: (0, 0)),
            pl.BlockSpec((1, TN, D),
                         lambda e, s: (jnp.where(s < NT, e, jnp.minimum(e + 1, E - 1)),
                                       jnp.where(s < NT, s, 0), 0)),
            pl.BlockSpec((1, TN, D),
                         lambda e, s: (jnp.where(s < NT, e, jnp.minimum(e + 1, E - 1)),
                                       NT + jnp.where(s < NT, s, 0), 0)),
            pl.BlockSpec((1, TND, D),
                         lambda e, s: (jnp.where(s >= NT - 1, e, jnp.maximum(e - 1, 0)),
                                       jnp.where(s >= NT - 1,
                                                 jnp.clip(s - NT, 0, ND - 1),
                                                 jnp.where(e > 0, ND - 1, 0)), 0)),
        ],
        out_specs=pl.BlockSpec((1, BT, TND), lambda e, s: (e, 0, jnp.maximum(s - NT, 0))),
        out_shape=jax.ShapeDtypeStruct((E, BT, D), jnp.float32),
        scratch_shapes=[pltpu.VMEM((BT, I), jnp.float32)],
        compiler_params=pltpu.CompilerParams(vmem_limit_bytes=56 * 1024 * 1024),
    )(x, ert, rw, w13_weight, w13_weight, w2_weight)

    keys = ert.reshape(-1)
    return _combine(ysc.reshape(E * BT, D), keys)


# speedup vs baseline: 1.0298x; 1.0298x over previous
import functools

import jax
import jax.numpy as jnp
from jax import lax
from jax.experimental import pallas as pl
from jax.experimental.pallas import tpu as pltpu
from jax.experimental.pallas import tpu_sc as plsc

BT = 256
E = 8
D = 2048
I = 2048
TOPK = 2
NP = BT * TOPK
TN = 512
NT = I // TN
TND = 512
ND = D // TND
S = NT + ND

NTILES = 32
TPT = BT // NTILES

_MESH = plsc.VectorSubcoreMesh(core_axis_name="c", subcore_axis_name="s")


def _wid():
    return lax.axis_index("s") * 2 + lax.axis_index("c")


def _k1_body(x_ref, wu_ref, wg_ref, h_ref):
    x = x_ref[...]
    u = jax.lax.dot_general(x, wu_ref[0], (((1,), (1,)), ((), ())),
                            preferred_element_type=jnp.float32)
    g = jax.lax.dot_general(x, wg_ref[0], (((1,), (1,)), ((), ())),
                            preferred_element_type=jnp.float32)
    h_ref[0] = (u * jax.nn.sigmoid(u)) * g


def _k2_body(ert_ref, rw_ref, h_ref, w2_ref, y_ref):
    e = pl.program_id(0)
    y = jax.lax.dot_general(h_ref[0], w2_ref[0], (((1,), (1,)), ((), ())),
                            preferred_element_type=jnp.float32)
    sel = (ert_ref[...] == e).astype(jnp.float32) * rw_ref[...]
    we = jnp.sum(sel, axis=1, keepdims=True)
    y_ref[0] = y * we


@functools.partial(
    pl.kernel,
    out_type=jax.ShapeDtypeStruct((BT, D), jnp.float32),
    mesh=_MESH,
    compiler_params=pltpu.CompilerParams(needs_layout_passes=False),
    scratch_types=[
        pltpu.VMEM((16,), jnp.int32),
        pltpu.VMEM((TPT,), jnp.int32),
        pltpu.VMEM((TPT,), jnp.int32),
        pltpu.VMEM((TPT, D), jnp.float32),
        pltpu.VMEM((TPT, D), jnp.float32),
        pltpu.SemaphoreType.DMA,
        pltpu.SemaphoreType.DMA,
    ],
)
def _combine(ysc_hbm, keys_hbm, out_hbm, k_v, ia_v, ib_v, ra_v, rb_v, sa, sb):
    base = _wid() * TPT
    pltpu.sync_copy(keys_hbm.at[pl.ds(base * TOPK, 16)], k_v)
    iota = lax.iota(jnp.int32, 16)
    ones = jnp.ones((16,), jnp.int32)
    half = lax.shift_right_logical(iota, ones)
    tok = jnp.full((16,), base, jnp.int32) + half
    pc = k_v[...] * jnp.full((16,), BT, jnp.int32) + tok
    even = jnp.bitwise_and(iota, ones) == 0
    plsc.store_scatter(ia_v, [half], pc, mask=even)
    plsc.store_scatter(ib_v, [half], pc, mask=jnp.logical_not(even))
    ca = pltpu.async_copy(ysc_hbm.at[ia_v], ra_v, sa)
    cb = pltpu.async_copy(ysc_hbm.at[ib_v], rb_v, sb)
    ca.wait()
    cb.wait()

    def add_body(j, _):
        sl = pl.ds(j * 16, 16)
        for r in range(TPT):
            ra_v[r, sl] = ra_v[r, sl] + rb_v[r, sl]
        return 0

    lax.fori_loop(0, D // 16, add_body, 0)
    pltpu.sync_copy(ra_v, out_hbm.at[pl.ds(base, TPT)])


def kernel(hidden_states, expert_routing_table, router_weights, w13_weight, w2_weight):
    x = hidden_states.astype(jnp.float32)
    ert = expert_routing_table.astype(jnp.int32)
    rw = router_weights.astype(jnp.float32)

    h = pl.pallas_call(
        _k1_body,
        grid=(E, NT),
        in_specs=[
            pl.BlockSpec((BT, D), lambda e, n: (0, 0)),
            pl.BlockSpec((1, TN, D), lambda e, n: (e, n, 0)),
            pl.BlockSpec((1, TN, D), lambda e, n: (e, n + NT, 0)),
        ],
        out_specs=pl.BlockSpec((1, BT, TN), lambda e, n: (e, 0, n)),
        out_shape=jax.ShapeDtypeStruct((E, BT, I), jnp.float32),
    )(x, w13_weight, w13_weight)

    ysc = pl.pallas_call(
        _k2_body,
        grid=(E, ND),
        in_specs=[
            pl.BlockSpec((BT, 2), lambda e, n: (0, 0)),
            pl.BlockSpec((BT, 2), lambda e, n: (0, 0)),
            pl.BlockSpec((1, BT, I), lambda e, n: (e, 0, 0)),
            pl.BlockSpec((1, TND, I), lambda e, n: (e, n, 0)),
        ],
        out_specs=pl.BlockSpec((1, BT, TND), lambda e, n: (e, 0, n)),
        out_shape=jax.ShapeDtypeStruct((E, BT, D), jnp.float32),
    )(ert, rw, h, w2_weight)

    keys = ert.reshape(-1)
    return _combine(ysc.reshape(E * BT, D), keys)


# speedup vs baseline: 1.0592x; 1.0285x over previous
import functools

import jax
import jax.numpy as jnp
from jax import lax
from jax.experimental import pallas as pl
from jax.experimental.pallas import tpu as pltpu
from jax.experimental.pallas import tpu_sc as plsc

BT = 256
E = 8
D = 2048
I = 2048
TOPK = 2
NP = BT * TOPK
TN = 512
NT = I // TN
TND = 512
ND = D // TND
S = NT + ND

NTILES = 32
TPT = BT // NTILES

_MESH = plsc.VectorSubcoreMesh(core_axis_name="c", subcore_axis_name="s")


def _wid():
    return lax.axis_index("s") * 2 + lax.axis_index("c")


def _k1_body(x_ref, wu_ref, wg_ref, h_ref):
    x = x_ref[...]
    u = jax.lax.dot_general(x, wu_ref[0], (((1,), (1,)), ((), ())),
                            preferred_element_type=jnp.float32)
    g = jax.lax.dot_general(x, wg_ref[0], (((1,), (1,)), ((), ())),
                            preferred_element_type=jnp.float32)
    h_ref[0] = ((u * jax.nn.sigmoid(u)) * g).astype(jnp.bfloat16)


def _k2_body(ert_ref, rw_ref, h_ref, w2_ref, y_ref):
    e = pl.program_id(0)
    y = jax.lax.dot_general(h_ref[0].astype(jnp.float32), w2_ref[0],
                            (((1,), (1,)), ((), ())),
                            preferred_element_type=jnp.float32)
    sel = (ert_ref[...] == e).astype(jnp.float32) * rw_ref[...]
    we = jnp.sum(sel, axis=1, keepdims=True)
    y_ref[...] = y * we


@functools.partial(
    pl.kernel,
    out_type=jax.ShapeDtypeStruct((BT, D), jnp.float32),
    mesh=_MESH,
    compiler_params=pltpu.CompilerParams(needs_layout_passes=False),
    scratch_types=[
        pltpu.VMEM((16,), jnp.int32),
        pltpu.VMEM((TPT,), jnp.int32),
        pltpu.VMEM((TPT,), jnp.int32),
        pltpu.VMEM((TPT, D), jnp.float32),
        pltpu.VMEM((TPT, D), jnp.float32),
        pltpu.SemaphoreType.DMA,
        pltpu.SemaphoreType.DMA,
    ],
)
def _combine(ysc_hbm, keys_hbm, out_hbm, k_v, ia_v, ib_v, ra_v, rb_v, sa, sb):
    base = _wid() * TPT
    pltpu.sync_copy(keys_hbm.at[pl.ds(base * TOPK, 16)], k_v)
    iota = lax.iota(jnp.int32, 16)
    ones = jnp.ones((16,), jnp.int32)
    half = lax.shift_right_logical(iota, ones)
    tok = jnp.full((16,), base, jnp.int32) + half
    pc = k_v[...] * jnp.full((16,), BT, jnp.int32) + tok
    even = jnp.bitwise_and(iota, ones) == 0
    plsc.store_scatter(ia_v, [half], pc, mask=even)
    plsc.store_scatter(ib_v, [half], pc, mask=jnp.logical_not(even))
    ca = pltpu.async_copy(ysc_hbm.at[ia_v], ra_v, sa)
    cb = pltpu.async_copy(ysc_hbm.at[ib_v], rb_v, sb)
    ca.wait()
    cb.wait()

    def add_body(j, _):
        sl = pl.ds(j * 16, 16)
        for r in range(TPT):
            ra_v[r, sl] = ra_v[r, sl] + rb_v[r, sl]
        return 0

    lax.fori_loop(0, D // 16, add_body, 0)
    pltpu.sync_copy(ra_v, out_hbm.at[pl.ds(base, TPT)])


def kernel(hidden_states, expert_routing_table, router_weights, w13_weight, w2_weight):
    x = hidden_states.astype(jnp.float32)
    ert = expert_routing_table.astype(jnp.int32)
    rw = router_weights.astype(jnp.float32)

    h = pl.pallas_call(
        _k1_body,
        grid=(E, NT),
        in_specs=[
            pl.BlockSpec((BT, D), lambda e, n: (0, 0)),
            pl.BlockSpec((1, TN, D), lambda e, n: (e, n, 0)),
            pl.BlockSpec((1, TN, D), lambda e, n: (e, n + NT, 0)),
        ],
        out_specs=pl.BlockSpec((1, BT, TN), lambda e, n: (e, 0, n)),
        out_shape=jax.ShapeDtypeStruct((E, BT, I), jnp.bfloat16),
    )(x, w13_weight, w13_weight)

    ysc = pl.pallas_call(
        _k2_body,
        grid=(E, ND),
        in_specs=[
            pl.BlockSpec((BT, 2), lambda e, n: (0, 0)),
            pl.BlockSpec((BT, 2), lambda e, n: (0, 0)),
            pl.BlockSpec((1, BT, I), lambda e, n: (e, 0, 0)),
            pl.BlockSpec((1, TND, I), lambda e, n: (e, n, 0)),
        ],
        out_specs=pl.BlockSpec((BT, TND), lambda e, n: (e, n)),
        out_shape=jax.ShapeDtypeStruct((E * BT, D), jnp.float32),
    )(ert, rw, h, w2_weight)

    keys = ert.reshape(-1)
    return _combine(ysc, keys)
